# Initial kernel scaffold; baseline (speedup 1.0000x reference)
import jax
import jax.numpy as jnp
from jax import lax
from jax.experimental import pallas as pl
from jax.experimental.pallas import tpu as pltpu
from jax.experimental.pallas import tpu_sc as plsc

N_NODES = 10000
N_EDGES = 320000
D = 128
N_LAYERS = 3

NC = 2
NS = 16
NW = NC * NS
E_PER_W = N_EDGES // NW
CHUNK = 80
NCHUNK = E_PER_W // CHUNK
ROWS_PER_TILE = N_NODES // NS
ZROWS = 125


def _sc_layer_body(table_hbm, src_hbm, dst_hbm, w_hbm, out_hbm,
                   src_v, dst_v, w_v, rows_v, zbuf_v, acc_sh, sem):
  c = lax.axis_index("c")
  s = lax.axis_index("s")
  wid = c * NS + s

  zeros16 = jnp.zeros((16,), jnp.float32)

  def zero_row(r, carry):
    for col in range(D // 16):
      zbuf_v[r, pl.ds(col * 16, 16)] = zeros16
    return carry

  lax.fori_loop(0, ZROWS, zero_row, 0)
  for t in range(ROWS_PER_TILE // ZROWS):
    pltpu.sync_copy(zbuf_v, acc_sh.at[pl.ds(s * ROWS_PER_TILE + t * ZROWS,
                                            ZROWS)])
  plsc.subcore_barrier()

  pltpu.sync_copy(src_hbm.at[wid], src_v)
  pltpu.sync_copy(dst_hbm.at[wid], dst_v)
  pltpu.sync_copy(w_hbm.at[wid], w_v)

  def chunk_body(j, carry):
    pltpu.async_copy(table_hbm.at[src_v.at[j]], rows_v, sem).wait()

    def edge_body(k, kcarry):
      jidx = jnp.full((16,), j, jnp.int32)
      kidx = jnp.full((16,), k, jnp.int32)
      wb = plsc.load_gather(w_v, [jidx, kidx])
      for col in range(D // 16):
        x = rows_v[k, pl.ds(col * 16, 16)]
        rows_v[k, pl.ds(col * 16, 16)] = x * wb
      return kcarry

    lax.fori_loop(0, CHUNK, edge_body, 0)

    pltpu.sync_copy(rows_v, acc_sh.at[dst_v.at[j]], add=True)
    return carry

  lax.fori_loop(0, NCHUNK, chunk_body, 0)
  plsc.subcore_barrier()

  for t in range(ROWS_PER_TILE // ZROWS):
    r0 = s * ROWS_PER_TILE + t * ZROWS
    pltpu.sync_copy(acc_sh.at[pl.ds(r0, ZROWS)],
                    out_hbm.at[c, pl.ds(r0, ZROWS)])


_sc_layer = pl.kernel(
    _sc_layer_body,
    out_type=jax.ShapeDtypeStruct((NC, N_NODES, D), jnp.float32),
    mesh=plsc.VectorSubcoreMesh(core_axis_name="c", subcore_axis_name="s"),
    scratch_types=[
        pltpu.VMEM((NCHUNK, CHUNK), jnp.int32),
        pltpu.VMEM((NCHUNK, CHUNK), jnp.int32),
        pltpu.VMEM((NCHUNK, CHUNK), jnp.float32),
        pltpu.VMEM((CHUNK, D), jnp.float32),
        pltpu.VMEM((ZROWS, D), jnp.float32),
        pltpu.VMEM_SHARED((N_NODES, D), jnp.float32),
        pltpu.SemaphoreType.DMA,
    ],
)


def _matmul_body(f_ref, w_ref, b_ref, o_ref):
  o_ref[...] = (
      jnp.dot(f_ref[...], w_ref[...], preferred_element_type=jnp.float32)
      + b_ref[...]
  )


def _matmul(feats, W, b2):
  grid = 20
  rb = N_NODES // grid
  return pl.pallas_call(
      _matmul_body,
      grid=(grid,),
      in_specs=[
          pl.BlockSpec((rb, D), lambda i: (i, 0)),
          pl.BlockSpec((D, D), lambda i: (0, 0)),
          pl.BlockSpec((1, D), lambda i: (0, 0)),
      ],
      out_specs=pl.BlockSpec((rb, D), lambda i: (i, 0)),
      out_shape=jax.ShapeDtypeStruct((N_NODES, D), jnp.float32),
  )(feats, W, b2)


def _combine_body(p_ref, o_ref):
  o_ref[...] = p_ref[0] + p_ref[1]


def _combine(p):
  grid = 10
  rb = N_NODES // grid
  return pl.pallas_call(
      _combine_body,
      grid=(grid,),
      in_specs=[pl.BlockSpec((NC, rb, D), lambda i: (0, i, 0))],
      out_specs=pl.BlockSpec((rb, D), lambda i: (i, 0)),
      out_shape=jax.ShapeDtypeStruct((N_NODES, D), jnp.float32),
  )(p)


def _mean_body(e0_ref, e1_ref, e2_ref, p3_ref, o_ref):
  s = e0_ref[...] + e1_ref[...] + e2_ref[...] + p3_ref[0] + p3_ref[1]
  o_ref[...] = s * 0.25


def _mean(e0, e1, e2, p3):
  grid = 10
  rb = N_NODES // grid
  return pl.pallas_call(
      _mean_body,
      grid=(grid,),
      in_specs=[
          pl.BlockSpec((rb, D), lambda i: (i, 0)),
          pl.BlockSpec((rb, D), lambda i: (i, 0)),
          pl.BlockSpec((rb, D), lambda i: (i, 0)),
          pl.BlockSpec((NC, rb, D), lambda i: (0, i, 0)),
      ],
      out_specs=pl.BlockSpec((rb, D), lambda i: (i, 0)),
      out_shape=jax.ShapeDtypeStruct((N_NODES, D), jnp.float32),
  )(e0, e1, e2, p3)


@jax.jit
def kernel(feats, edge_index, edge_weight, W, b):
  dst = edge_index[0].reshape(NW, NCHUNK, CHUNK)
  src = edge_index[1].reshape(NW, NCHUNK, CHUNK)
  w = edge_weight.reshape(NW, NCHUNK, CHUNK)

  e0 = _matmul(feats, W, b.reshape(1, D))
  p1 = _sc_layer(e0, src, dst, w)
  e1 = _combine(p1)
  p2 = _sc_layer(e1, src, dst, w)
  e2 = _combine(p2)
  p3 = _sc_layer(e2, src, dst, w)
  return _mean(e0, e1, e2, p3)


# speedup vs baseline: 3.6145x; 3.6145x over previous
import functools

import jax
import jax.numpy as jnp
from jax import lax
from jax.experimental import pallas as pl
from jax.experimental.pallas import tpu as pltpu
from jax.experimental.pallas import tpu_sc as plsc

N_NODES = 10000
N_EDGES = 320000
D = 128
N_LAYERS = 3

NC = 2
NS = 16
DH = D // NC
E_PER_T = N_EDGES // NS
CHUNK = 80
NCHUNK = E_PER_T // CHUNK
ROWS_PER_TILE = N_NODES // NS
ZROWS = 125


def _sc_layer_body(table_hbm, src_hbm, dst_hbm, w_hbm, out_hbm,
                   src_v, dst_v, w_v, rows_v, zbuf_v, acc_sh, sem):
  c = lax.axis_index("c")
  s = lax.axis_index("s")

  zeros16 = jnp.zeros((16,), jnp.float32)

  def zero_row(r, carry):
    for col in range(DH // 16):
      zbuf_v[r, pl.ds(col * 16, 16)] = zeros16
    return carry

  lax.fori_loop(0, ZROWS, zero_row, 0)
  for t in range(ROWS_PER_TILE // ZROWS):
    pltpu.sync_copy(zbuf_v, acc_sh.at[pl.ds(s * ROWS_PER_TILE + t * ZROWS,
                                            ZROWS)])
  plsc.subcore_barrier()

  pltpu.sync_copy(src_hbm.at[s], src_v)
  pltpu.sync_copy(dst_hbm.at[s], dst_v)
  pltpu.sync_copy(w_hbm.at[s], w_v)

  def chunk_body(j, carry):
    pltpu.async_copy(table_hbm.at[c].at[src_v.at[j]], rows_v, sem).wait()

    def edge_body(k, kcarry):
      jidx = jnp.full((16,), j, jnp.int32)
      kidx = jnp.full((16,), k, jnp.int32)
      wb = plsc.load_gather(w_v, [jidx, kidx])
      for col in range(DH // 16):
        x = rows_v[k, pl.ds(col * 16, 16)]
        rows_v[k, pl.ds(col * 16, 16)] = x * wb
      return kcarry

    lax.fori_loop(0, CHUNK, edge_body, 0)

    pltpu.sync_copy(rows_v, acc_sh.at[dst_v.at[j]], add=True)
    return carry

  lax.fori_loop(0, NCHUNK, chunk_body, 0)
  plsc.subcore_barrier()

  for t in range(ROWS_PER_TILE // ZROWS):
    r0 = s * ROWS_PER_TILE + t * ZROWS
    pltpu.sync_copy(acc_sh.at[pl.ds(r0, ZROWS)],
                    out_hbm.at[c, pl.ds(r0, ZROWS)])


@functools.cache
def _get_sc_layer():
  return pl.kernel(
      _sc_layer_body,
      out_type=jax.ShapeDtypeStruct((NC, N_NODES, DH), jnp.float32),
      mesh=plsc.VectorSubcoreMesh(core_axis_name="c", subcore_axis_name="s",
                                  num_cores=NC, num_subcores=NS),
      scratch_types=[
          pltpu.VMEM((NCHUNK, CHUNK), jnp.int32),
          pltpu.VMEM((NCHUNK, CHUNK), jnp.int32),
          pltpu.VMEM((NCHUNK, CHUNK), jnp.float32),
          pltpu.VMEM((CHUNK, DH), jnp.float32),
          pltpu.VMEM((ZROWS, DH), jnp.float32),
          pltpu.VMEM_SHARED((N_NODES, DH), jnp.float32),
          pltpu.SemaphoreType.DMA,
      ],
      compiler_params=pltpu.CompilerParams(use_tc_tiling_on_sc=False,
                                           needs_layout_passes=False),
  )


def _matmul_body(f_ref, w_ref, b_ref, o_ref):
  r = (jnp.dot(f_ref[...], w_ref[...], preferred_element_type=jnp.float32)
       + b_ref[...])
  o_ref[0] = r[:, :DH]
  o_ref[1] = r[:, DH:]


def _matmul(feats, W, b2):
  grid = 10
  rb = N_NODES // grid
  return pl.pallas_call(
      _matmul_body,
      grid=(grid,),
      in_specs=[
          pl.BlockSpec((rb, D), lambda i: (i, 0)),
          pl.BlockSpec((D, D), lambda i: (0, 0)),
          pl.BlockSpec((1, D), lambda i: (0, 0)),
      ],
      out_specs=pl.BlockSpec((NC, rb, DH), lambda i: (0, i, 0)),
      out_shape=jax.ShapeDtypeStruct((NC, N_NODES, DH), jnp.float32),
  )(feats, W, b2)


def _mean_body(e0_ref, e1_ref, e2_ref, e3_ref, o_ref):
  s = e0_ref[...] + e1_ref[...] + e2_ref[...] + e3_ref[...]
  s = s * 0.25
  o_ref[...] = jnp.concatenate([s[0], s[1]], axis=-1)


def _mean(e0, e1, e2, e3):
  grid = 10
  rb = N_NODES // grid
  spec = pl.BlockSpec((NC, rb, DH), lambda i: (0, i, 0))
  return pl.pallas_call(
      _mean_body,
      grid=(grid,),
      in_specs=[spec, spec, spec, spec],
      out_specs=pl.BlockSpec((rb, D), lambda i: (i, 0)),
      out_shape=jax.ShapeDtypeStruct((N_NODES, D), jnp.float32),
  )(e0, e1, e2, e3)


@jax.jit
def kernel(feats, edge_index, edge_weight, W, b):
  dst = edge_index[0].reshape(NS, NCHUNK, CHUNK)
  src = edge_index[1].reshape(NS, NCHUNK, CHUNK)
  w = edge_weight.reshape(NS, NCHUNK, CHUNK)

  sc_layer = _get_sc_layer()
  e0 = _matmul(feats, W, b.reshape(1, D))
  e1 = sc_layer(e0, src, dst, w)
  e2 = sc_layer(e1, src, dst, w)
  e3 = sc_layer(e2, src, dst, w)
  return _mean(e0, e1, e2, e3)
